# Initial kernel scaffold; baseline (speedup 1.0000x reference)
import functools

import jax
import jax.numpy as jnp
from jax import lax
from jax.experimental import pallas as pl
from jax.experimental.pallas import tpu as pltpu
from jax.experimental.pallas import tpu_sc as plsc

NY, NX = 512, 512
NW = 32
CHUNK = 128


def _sc_scatter(pf_flat, idx_global, canvas0):
    n, c = pf_flat.shape
    rows_total = canvas0.shape[0]
    per_w = n // NW
    n_chunks = per_w // CHUNK

    mesh = plsc.VectorSubcoreMesh(core_axis_name="c", subcore_axis_name="s")

    @functools.partial(
        pl.kernel,
        mesh=mesh,
        out_type=jax.ShapeDtypeStruct((rows_total, c), jnp.float32),
        scratch_types=[
            pltpu.VMEM((CHUNK,), jnp.int32),
            pltpu.VMEM((CHUNK, c), jnp.float32),
            pltpu.SemaphoreType.DMA,
        ],
        input_output_aliases={2: 0},
    )
    def scatter_kernel(pf_hbm, idx_hbm, canvas_in, out_hbm, idx_v, rows_v, sem):
        del canvas_in
        wid = lax.axis_index("s") * 2 + lax.axis_index("c")
        base = wid * per_w

        def body(i, carry):
            off = pl.multiple_of(base + i * CHUNK, CHUNK)
            pltpu.sync_copy(idx_hbm.at[pl.ds(off, CHUNK)], idx_v)
            pltpu.sync_copy(pf_hbm.at[pl.ds(off, CHUNK)], rows_v)
            pltpu.async_copy(rows_v, out_hbm.at[idx_v], sem).wait()
            return carry

        lax.fori_loop(0, n_chunks, body, 0)

    return scatter_kernel(pf_flat, idx_global, canvas0)


def _tc_transpose(canvas_nhwc):
    b, _, c = canvas_nhwc.shape
    rows = 8

    def body(in_ref, out_ref):
        x = in_ref[0]
        out_ref[0] = x.reshape(rows, NX, c).transpose(2, 0, 1)

    return pl.pallas_call(
        body,
        grid=(b, NY // rows),
        in_specs=[
            pl.BlockSpec((1, rows * NX, c), lambda i, j: (i, j, 0)),
        ],
        out_specs=pl.BlockSpec((1, c, rows, NX), lambda i, j: (i, 0, j, 0)),
        out_shape=jax.ShapeDtypeStruct((b, c, NY, NX), jnp.float32),
    )(canvas_nhwc)


@jax.jit
def kernel(pillar_features, coords):
    b, p, c = pillar_features.shape
    y = coords[:, :, 2].astype(jnp.int32)
    x = coords[:, :, 3].astype(jnp.int32)
    idx_global = (
        jnp.arange(b, dtype=jnp.int32)[:, None] * (NY * NX) + y * NX + x
    ).reshape(-1)
    pf_flat = pillar_features.reshape(b * p, c)
    canvas0 = jnp.zeros((b * NY * NX, c), jnp.float32)
    flat = _sc_scatter(pf_flat, idx_global, canvas0)
    return _tc_transpose(flat.reshape(b, NY * NX, c))


# speedup vs baseline: 25.5789x; 25.5789x over previous
import functools

import jax
import jax.numpy as jnp
from jax import lax
from jax.experimental import pallas as pl
from jax.experimental.pallas import tpu as pltpu
from jax.experimental.pallas import tpu_sc as plsc

NY, NX = 512, 512
NW = 32
CHUNK = 128
WIDE = 128


def _sc_scatter(pf_flat, idx_flat, idx_2d):
    n, _ = pf_flat.shape
    nb = n // 32768
    rows_total = nb * NY * NX
    per_w = n // NW
    n_sub = per_w // CHUNK
    p = n // nb
    pix_w = rows_total // NW
    rows_w = pix_w // NX
    sub_per_b = NW // nb

    mesh = plsc.VectorSubcoreMesh(core_axis_name="c", subcore_axis_name="s")

    @functools.partial(
        pl.kernel,
        mesh=mesh,
        out_type=(
            jax.ShapeDtypeStruct((nb, NY, NX), jnp.int32),
            jax.ShapeDtypeStruct((rows_total, WIDE), jnp.float32),
        ),
        scratch_types=[
            pltpu.VMEM((rows_w, NX), jnp.int32),
            pltpu.VMEM((p,), jnp.int32),
            pltpu.VMEM((n_sub, CHUNK), jnp.int32),
            pltpu.VMEM((CHUNK, WIDE), jnp.float32),
            pltpu.SemaphoreType.DMA,
        ],
        compiler_params=pltpu.CompilerParams(needs_layout_passes=False),
    )
    def scatter_kernel(pf_hbm, idxf_hbm, idx2_hbm, mask_hbm, out_hbm,
                       mask_v, bidx_v, sidx_v, rows_v, sem):
        wid = lax.axis_index("s") * 2 + lax.axis_index("c")
        batch = wid // sub_per_b
        pix_base = wid * pix_w

        zeros16 = jnp.zeros((16,), jnp.int32)
        ones16 = jnp.ones((16,), jnp.int32)

        def zero_body(i, carry):
            r = i // (NX // 16)
            j = i % (NX // 16)
            mask_v[r, pl.ds(j * 16, 16)] = zeros16
            return carry

        lax.fori_loop(0, pix_w // 16, zero_body, 0)

        pltpu.sync_copy(idxf_hbm.at[pl.ds(batch * p, p)], bidx_v)

        def mask_body(i, carry):
            v = bidx_v[pl.ds(i * 16, 16)]
            pos = v - pix_base
            m = (pos >= 0) & (pos < pix_w)
            plsc.store_scatter(
                mask_v,
                [lax.shift_right_logical(pos, 9), pos & (NX - 1)],
                ones16,
                mask=m,
            )
            return carry

        lax.fori_loop(0, p // 16, mask_body, 0)
        pltpu.sync_copy(
            mask_v, mask_hbm.at[batch, pl.ds((wid % sub_per_b) * rows_w, rows_w)]
        )

        pltpu.sync_copy(idx2_hbm.at[pl.ds(wid * n_sub, n_sub)], sidx_v)

        def scat_body(j, carry):
            off = pl.multiple_of(wid * per_w + j * CHUNK, CHUNK)
            pltpu.sync_copy(pf_hbm.at[pl.ds(off, CHUNK)], rows_v)
            pltpu.async_copy(rows_v, out_hbm.at[sidx_v.at[j]], sem).wait()
            return carry

        lax.fori_loop(0, n_sub, scat_body, 0)

    return scatter_kernel(pf_flat, idx_flat, idx_2d)


def _tc_transpose(mask_img, canvas_nhwc, c):
    b = canvas_nhwc.shape[0]
    rows = 8

    def body(mask_ref, in_ref, out_ref):
        m = mask_ref[0] != 0
        x = in_ref[0][:, :c]
        xt = x.reshape(rows, NX, c).transpose(2, 0, 1)
        out_ref[0] = jnp.where(m[None], xt, jnp.float32(0.0))

    return pl.pallas_call(
        body,
        grid=(b, NY // rows),
        in_specs=[
            pl.BlockSpec((1, rows, NX), lambda i, j: (i, j, 0)),
            pl.BlockSpec((1, rows * NX, WIDE), lambda i, j: (i, j, 0)),
        ],
        out_specs=pl.BlockSpec((1, c, rows, NX), lambda i, j: (i, 0, j, 0)),
        out_shape=jax.ShapeDtypeStruct((b, c, NY, NX), jnp.float32),
    )(mask_img, canvas_nhwc)


@jax.jit
def kernel(pillar_features, coords):
    b, p, c = pillar_features.shape
    y = coords[:, :, 2].astype(jnp.int32)
    x = coords[:, :, 3].astype(jnp.int32)
    idx_global = (
        jnp.arange(b, dtype=jnp.int32)[:, None] * (NY * NX) + y * NX + x
    ).reshape(-1)
    pf_flat = jnp.pad(
        pillar_features.reshape(b * p, c), ((0, 0), (0, WIDE - c))
    )
    mask, flat = _sc_scatter(
        pf_flat, idx_global, idx_global.reshape(-1, CHUNK)
    )
    return _tc_transpose(mask, flat.reshape(b, NY * NX, WIDE), c)
